# baseline (speedup 1.0000x reference)
import functools

import jax
import jax.numpy as jnp
from jax import lax
from jax.experimental import pallas as pl
from jax.experimental.pallas import tpu as pltpu
from jax.experimental.pallas import tpu_sc as plsc

H_TGT = 680
W_TGT = 680
NSUB = 16
TILE_PTS = 4608
ROW_STRIDE = 768
NBINS_PAD = H_TGT * ROW_STRIDE
STRIPE = NBINS_PAD // NSUB
NCH_OUT = 6


def _idx_body(n_blk, n_valid, n_grid, uv_ref, s_ref, o_ref):
    j = pl.program_id(0)
    ws = s_ref[0, 0]
    hs = s_ref[1, 0]
    u = uv_ref[0]
    v = uv_ref[1]
    ui = jnp.clip(jnp.round(u * ws).astype(jnp.int32), 0, W_TGT - 1)
    vi = jnp.clip(jnp.round(v * hs).astype(jnp.int32), 0, H_TGT - 1)
    flat = vi * ROW_STRIDE + ui

    @pl.when(j < n_grid - 1)
    def _():
        o_ref[...] = flat

    @pl.when(j == n_grid - 1)
    def _():
        pos = (j * n_blk
               + lax.broadcasted_iota(jnp.int32, flat.shape, 0) * 128
               + lax.broadcasted_iota(jnp.int32, flat.shape, 1))
        trash = (pos & 255) * ROW_STRIDE + W_TGT
        o_ref[...] = jnp.where(pos < n_valid, flat, trash)


def _sc_histogram(feats, idx, zeros_seg, ones_tile, n_tiles):
    pts_per_sub = n_tiles * TILE_PTS
    n_pad = NSUB * pts_per_sub
    half = n_tiles // 2
    mesh = plsc.VectorSubcoreMesh(core_axis_name="c", subcore_axis_name="s")

    vmem_f = pltpu.VMEM((TILE_PTS,), jnp.float32)
    vmem_i = pltpu.VMEM((TILE_PTS,), jnp.int32)

    @functools.partial(
        pl.kernel,
        out_type=jax.ShapeDtypeStruct((NCH_OUT * NBINS_PAD,), jnp.float32),
        mesh=mesh,
        compiler_params=pltpu.CompilerParams(needs_layout_passes=False),
        scratch_types=[
            vmem_f, vmem_f, vmem_i,
            vmem_f, vmem_f, vmem_i,
            vmem_f,
            pltpu.VMEM_SHARED((NBINS_PAD,), jnp.float32),
            pltpu.VMEM_SHARED((NBINS_PAD,), jnp.float32),
            pltpu.VMEM_SHARED((NBINS_PAD,), jnp.float32),
            pltpu.SemaphoreType.DMA,
            pltpu.SemaphoreType.DMA,
            pltpu.SemaphoreType.DMA,
            pltpu.SemaphoreType.DMA,
            pltpu.SemaphoreType.DMA,
            pltpu.SemaphoreType.DMA,
            pltpu.SemaphoreType.DMA,
            pltpu.SemaphoreType.DMA,
            pltpu.SemaphoreType.DMA,
            pltpu.SemaphoreType.DMA,
            pltpu.SemaphoreType.DMA,
            pltpu.SemaphoreType.DMA,
        ],
    )
    def hist_kernel(feats_hbm, idx_hbm, zeros_hbm, ones_hbm, out_hbm,
                    f0_0, f1_0, idx_0,
                    f0_1, f1_1, idx_1,
                    ones_v, hist_a, hist_b, hist_c,
                    sa_0, sb_0, sc_0, sa_1, sb_1, sc_1,
                    ta_0, tb_0, tc_0, ta_1, tb_1, tc_1):
        c = lax.axis_index("c")
        s = lax.axis_index("s")
        f0s, f1s, idxs = [f0_0, f0_1], [f1_0, f1_1], [idx_0, idx_1]
        sas, sbs, scs = [sa_0, sa_1], [sb_0, sb_1], [sc_0, sc_1]
        tas, tbs, tcs = [ta_0, ta_1], [tb_0, tb_1], [tc_0, tc_1]

        stripe = pl.ds(s * STRIPE, STRIPE)
        ch_f0 = 2 * c
        ch_f1 = 2 * c + 1

        pending = {}

        pending_stage = {}

        def stage(t):
            b = t % 2
            base = s * pts_per_sub + t * TILE_PTS
            pending_stage[b] = [
                pltpu.async_copy(idx_hbm.at[pl.ds(base, TILE_PTS)],
                                 idxs[b], tas[b]),
                pltpu.async_copy(feats_hbm.at[pl.ds(ch_f0 * n_pad + base,
                                                    TILE_PTS)],
                                 f0s[b], tbs[b]),
                pltpu.async_copy(feats_hbm.at[pl.ds(ch_f1 * n_pad + base,
                                                    TILE_PTS)],
                                 f1s[b], tcs[b]),
            ]

        def wait_stage(b):
            for d in pending_stage.pop(b, []):
                d.wait()

        def fire(t):
            b = t % 2
            descs = []
            count_core = "c0" if t < half else "c1"

            @pl.when(c == (0 if t < half else 1))
            def _():
                descs.append((count_core, pltpu.async_copy(
                    ones_v, hist_a.at[idxs[b]], sas[b], add=True)))

            descs.append((None, pltpu.async_copy(
                f0s[b], hist_b.at[idxs[b]], sbs[b], add=True)))
            descs.append((None, pltpu.async_copy(
                f1s[b], hist_c.at[idxs[b]], scs[b], add=True)))
            pending[b] = descs

        def drain(b):
            for cond, d in pending.get(b, []):
                if cond is None:
                    d.wait()
                elif cond == "c0":
                    @pl.when(c == 0)
                    def _():
                        d.wait()
                else:
                    @pl.when(c == 1)
                    def _():
                        d.wait()
            pending[b] = []

        stage(0)
        pltpu.sync_copy(zeros_hbm, hist_a.at[stripe])
        pltpu.sync_copy(zeros_hbm, hist_b.at[stripe])
        pltpu.sync_copy(zeros_hbm, hist_c.at[stripe])
        pltpu.sync_copy(ones_hbm, ones_v)
        plsc.subcore_barrier()

        for t in range(n_tiles):
            if t + 1 < n_tiles:
                drain((t + 1) % 2)
                stage(t + 1)
            wait_stage(t % 2)
            fire(t)
        drain(0)
        drain(1)

        plsc.subcore_barrier()

        @pl.when(c == 0)
        def _():
            pltpu.sync_copy(hist_a.at[stripe],
                            out_hbm.at[pl.ds(s * STRIPE, STRIPE)])
            pltpu.sync_copy(hist_b.at[stripe],
                            out_hbm.at[pl.ds(NBINS_PAD + s * STRIPE, STRIPE)])
            pltpu.sync_copy(hist_c.at[stripe],
                            out_hbm.at[pl.ds(2 * NBINS_PAD + s * STRIPE,
                                             STRIPE)])

        @pl.when(c == 1)
        def _():
            pltpu.sync_copy(hist_b.at[stripe],
                            out_hbm.at[pl.ds(3 * NBINS_PAD + s * STRIPE,
                                             STRIPE)])
            pltpu.sync_copy(hist_c.at[stripe],
                            out_hbm.at[pl.ds(4 * NBINS_PAD + s * STRIPE,
                                             STRIPE)])
            pltpu.sync_copy(hist_a.at[stripe],
                            out_hbm.at[pl.ds(5 * NBINS_PAD + s * STRIPE,
                                             STRIPE)])

    return hist_kernel(feats, idx, zeros_seg, ones_tile)


def _finalize_body(blk, h_ref, o_ref):
    def ch(k):
        return h_ref[k].reshape(blk, ROW_STRIDE)[:, :W_TGT]

    cts = ch(0) + ch(5)
    zero = cts == 0.0
    inv = jnp.where(zero, 0.0, 1.0 / jnp.where(zero, 1.0, cts))
    o_ref[0] = ch(1) * inv
    o_ref[1] = ch(2) * inv
    o_ref[2] = ch(3) * inv
    o_ref[3] = ch(4) * inv


def kernel(radar_points, original_image_size):
    n = radar_points.shape[0]
    n_tiles = -(-n // (NSUB * TILE_PTS))
    n_pad = NSUB * n_tiles * TILE_PTS

    h_orig = original_image_size[0].astype(jnp.float32)
    w_orig = original_image_size[1].astype(jnp.float32)
    w_scale = W_TGT / w_orig
    h_scale = H_TGT / h_orig

    rpp = jnp.pad(radar_points, ((0, n_pad - n), (0, 0)))
    uv = rpp[:, :2].T
    feats = rpp[:, 2:].T.reshape(-1)
    scales = jnp.stack([w_scale, h_scale]).reshape(2, 1)

    n_grid = 8
    n_blk = n_pad // n_grid
    blk_rows = n_blk // 128
    idx = pl.pallas_call(
        functools.partial(_idx_body, n_blk, n, n_grid),
        grid=(n_grid,),
        in_specs=[pl.BlockSpec((2, blk_rows, 128), lambda j: (0, j, 0)),
                  pl.BlockSpec((2, 1), lambda j: (0, 0))],
        out_specs=pl.BlockSpec((blk_rows, 128), lambda j: (j, 0)),
        out_shape=jax.ShapeDtypeStruct((n_pad // 128, 128), jnp.int32),
    )(uv.reshape(2, n_pad // 128, 128), scales).reshape(n_pad)

    zeros_seg = jnp.zeros((STRIPE,), jnp.float32)
    ones_tile = jnp.ones((TILE_PTS,), jnp.float32)

    hists = _sc_histogram(feats, idx, zeros_seg, ones_tile, n_tiles)

    blk = 40
    flat_rows = NBINS_PAD // 128
    return pl.pallas_call(
        functools.partial(_finalize_body, blk),
        grid=(H_TGT // blk,),
        in_specs=[pl.BlockSpec((NCH_OUT, blk * ROW_STRIDE // 128, 128),
                               lambda i: (0, i, 0))],
        out_specs=pl.BlockSpec((4, blk, W_TGT), lambda i: (0, i, 0)),
        out_shape=jax.ShapeDtypeStruct((4, H_TGT, W_TGT), jnp.float32),
    )(hists.reshape(NCH_OUT, flat_rows, 128))


# speedup vs baseline: 1.0269x; 1.0269x over previous
import functools

import jax
import jax.numpy as jnp
from jax import lax
from jax.experimental import pallas as pl
from jax.experimental.pallas import tpu as pltpu
from jax.experimental.pallas import tpu_sc as plsc

H_TGT = 680
W_TGT = 680
NSUB = 16
TILE_PTS = 4608
ROW_STRIDE = 768
NBINS_PAD = H_TGT * ROW_STRIDE
STRIPE = NBINS_PAD // NSUB
NCH_OUT = 6


def _idx_body(n_blk, n_valid, n_grid, uv_ref, s_ref, o_ref):
    j = pl.program_id(0)
    ws = s_ref[0, 0]
    hs = s_ref[1, 0]
    u = uv_ref[0]
    v = uv_ref[1]
    ui = jnp.clip(jnp.round(u * ws).astype(jnp.int32), 0, W_TGT - 1)
    vi = jnp.clip(jnp.round(v * hs).astype(jnp.int32), 0, H_TGT - 1)
    flat = vi * ROW_STRIDE + ui

    @pl.when(j < n_grid - 1)
    def _():
        o_ref[...] = flat

    @pl.when(j == n_grid - 1)
    def _():
        pos = (j * n_blk
               + lax.broadcasted_iota(jnp.int32, flat.shape, 0) * 128
               + lax.broadcasted_iota(jnp.int32, flat.shape, 1))
        trash = (pos & 255) * ROW_STRIDE + W_TGT
        o_ref[...] = jnp.where(pos < n_valid, flat, trash)


def _sc_histogram(feats, idx, zeros_seg, ones_tile, n_tiles):
    pts_per_sub = n_tiles * TILE_PTS
    n_pad = NSUB * pts_per_sub
    half = n_tiles // 2
    mesh = plsc.VectorSubcoreMesh(core_axis_name="c", subcore_axis_name="s")

    vmem_f = pltpu.VMEM((TILE_PTS,), jnp.float32)
    vmem_i = pltpu.VMEM((TILE_PTS,), jnp.int32)

    @functools.partial(
        pl.kernel,
        out_type=jax.ShapeDtypeStruct((NCH_OUT * NBINS_PAD,), jnp.float32),
        mesh=mesh,
        compiler_params=pltpu.CompilerParams(needs_layout_passes=False),
        scratch_types=[
            vmem_f, vmem_f, vmem_i,
            vmem_f, vmem_f, vmem_i,
            vmem_f,
            pltpu.VMEM_SHARED((NBINS_PAD,), jnp.float32),
            pltpu.VMEM_SHARED((NBINS_PAD,), jnp.float32),
            pltpu.VMEM_SHARED((NBINS_PAD,), jnp.float32),
            pltpu.SemaphoreType.DMA,
            pltpu.SemaphoreType.DMA,
            pltpu.SemaphoreType.DMA,
            pltpu.SemaphoreType.DMA,
            pltpu.SemaphoreType.DMA,
            pltpu.SemaphoreType.DMA,
            pltpu.SemaphoreType.DMA,
            pltpu.SemaphoreType.DMA,
            pltpu.SemaphoreType.DMA,
            pltpu.SemaphoreType.DMA,
            pltpu.SemaphoreType.DMA,
            pltpu.SemaphoreType.DMA,
        ],
    )
    def hist_kernel(feats_hbm, idx_hbm, zeros_hbm, ones_hbm, out_hbm,
                    f0_0, f1_0, idx_0,
                    f0_1, f1_1, idx_1,
                    ones_v, hist_a, hist_b, hist_c,
                    sa_0, sb_0, sc_0, sa_1, sb_1, sc_1,
                    ta_0, tb_0, tc_0, ta_1, tb_1, tc_1):
        c = lax.axis_index("c")
        s = lax.axis_index("s")
        f0s, f1s, idxs = [f0_0, f0_1], [f1_0, f1_1], [idx_0, idx_1]
        sas, sbs, scs = [sa_0, sa_1], [sb_0, sb_1], [sc_0, sc_1]
        tas, tbs, tcs = [ta_0, ta_1], [tb_0, tb_1], [tc_0, tc_1]

        stripe = pl.ds(s * STRIPE, STRIPE)
        ch_f0 = 2 * c
        ch_f1 = 2 * c + 1

        pending = {}

        pending_stage = {}

        def stage(t):
            b = t % 2
            base = s * pts_per_sub + t * TILE_PTS
            pending_stage[b] = [
                pltpu.async_copy(idx_hbm.at[pl.ds(base, TILE_PTS)],
                                 idxs[b], tas[b]),
                pltpu.async_copy(feats_hbm.at[pl.ds(ch_f0 * n_pad + base,
                                                    TILE_PTS)],
                                 f0s[b], tbs[b]),
                pltpu.async_copy(feats_hbm.at[pl.ds(ch_f1 * n_pad + base,
                                                    TILE_PTS)],
                                 f1s[b], tcs[b]),
            ]

        def wait_stage(b):
            for d in pending_stage.pop(b, []):
                d.wait()

        def fire(t):
            b = t % 2
            descs = []
            count_core = "c0" if t < half else "c1"

            @pl.when(c == (0 if t < half else 1))
            def _():
                descs.append((count_core, pltpu.async_copy(
                    ones_v, hist_a.at[idxs[b]], sas[b], add=True)))

            descs.append((None, pltpu.async_copy(
                f0s[b], hist_b.at[idxs[b]], sbs[b], add=True)))
            descs.append((None, pltpu.async_copy(
                f1s[b], hist_c.at[idxs[b]], scs[b], add=True)))
            pending[b] = descs

        def drain(b):
            for cond, d in pending.get(b, []):
                if cond is None:
                    d.wait()
                elif cond == "c0":
                    @pl.when(c == 0)
                    def _():
                        d.wait()
                else:
                    @pl.when(c == 1)
                    def _():
                        d.wait()
            pending[b] = []

        stage(0)
        pltpu.sync_copy(zeros_hbm, hist_a.at[stripe])
        pltpu.sync_copy(zeros_hbm, hist_b.at[stripe])
        pltpu.sync_copy(zeros_hbm, hist_c.at[stripe])
        pltpu.sync_copy(ones_hbm, ones_v)
        plsc.subcore_barrier()

        for t in range(n_tiles):
            if t + 1 < n_tiles:
                drain((t + 1) % 2)
                stage(t + 1)
            wait_stage(t % 2)
            fire(t)
        drain(0)
        drain(1)

        plsc.subcore_barrier()

        @pl.when(c == 0)
        def _():
            pltpu.sync_copy(hist_a.at[stripe],
                            out_hbm.at[pl.ds(s * STRIPE, STRIPE)])
            pltpu.sync_copy(hist_b.at[stripe],
                            out_hbm.at[pl.ds(NBINS_PAD + s * STRIPE, STRIPE)])
            pltpu.sync_copy(hist_c.at[stripe],
                            out_hbm.at[pl.ds(2 * NBINS_PAD + s * STRIPE,
                                             STRIPE)])

        @pl.when(c == 1)
        def _():
            pltpu.sync_copy(hist_b.at[stripe],
                            out_hbm.at[pl.ds(3 * NBINS_PAD + s * STRIPE,
                                             STRIPE)])
            pltpu.sync_copy(hist_c.at[stripe],
                            out_hbm.at[pl.ds(4 * NBINS_PAD + s * STRIPE,
                                             STRIPE)])
            pltpu.sync_copy(hist_a.at[stripe],
                            out_hbm.at[pl.ds(5 * NBINS_PAD + s * STRIPE,
                                             STRIPE)])

    return hist_kernel(feats, idx, zeros_seg, ones_tile)


def _finalize_body(blk, h_ref, o_ref):
    def ch(k):
        return h_ref[k].reshape(blk, ROW_STRIDE)[:, :W_TGT]

    cts = ch(0) + ch(5)
    zero = cts == 0.0
    inv = jnp.where(zero, 0.0, 1.0 / jnp.where(zero, 1.0, cts))
    o_ref[0] = ch(1) * inv
    o_ref[1] = ch(2) * inv
    o_ref[2] = ch(3) * inv
    o_ref[3] = ch(4) * inv


def kernel(radar_points, original_image_size):
    n = radar_points.shape[0]
    n_tiles = -(-n // (NSUB * TILE_PTS))
    n_pad = NSUB * n_tiles * TILE_PTS

    h_orig = original_image_size[0].astype(jnp.float32)
    w_orig = original_image_size[1].astype(jnp.float32)
    w_scale = W_TGT / w_orig
    h_scale = H_TGT / h_orig

    rpp = jnp.pad(radar_points, ((0, n_pad - n), (0, 0)))
    uv = rpp[:, :2].T
    feats = rpp[:, 2:].T.reshape(-1)
    scales = jnp.stack([w_scale, h_scale]).reshape(2, 1)

    n_grid = 8
    n_blk = n_pad // n_grid
    blk_rows = n_blk // 128
    idx = pl.pallas_call(
        functools.partial(_idx_body, n_blk, n, n_grid),
        grid=(n_grid,),
        in_specs=[pl.BlockSpec((2, blk_rows, 128), lambda j: (0, j, 0)),
                  pl.BlockSpec((2, 1), lambda j: (0, 0))],
        out_specs=pl.BlockSpec((blk_rows, 128), lambda j: (j, 0)),
        out_shape=jax.ShapeDtypeStruct((n_pad // 128, 128), jnp.int32),
    )(uv.reshape(2, n_pad // 128, 128), scales).reshape(n_pad)

    zeros_seg = jnp.zeros((STRIPE,), jnp.float32)
    ones_tile = jnp.ones((TILE_PTS,), jnp.float32)

    hists = _sc_histogram(feats, idx, zeros_seg, ones_tile, n_tiles)

    blk = 136
    flat_rows = NBINS_PAD // 128
    return pl.pallas_call(
        functools.partial(_finalize_body, blk),
        grid=(H_TGT // blk,),
        in_specs=[pl.BlockSpec((NCH_OUT, blk * ROW_STRIDE // 128, 128),
                               lambda i: (0, i, 0))],
        out_specs=pl.BlockSpec((4, blk, W_TGT), lambda i: (0, i, 0)),
        out_shape=jax.ShapeDtypeStruct((4, H_TGT, W_TGT), jnp.float32),
    )(hists.reshape(NCH_OUT, flat_rows, 128))
